# baseline (speedup 1.0000x reference)
import jax
import jax.numpy as jnp
import numpy as _np
from jax import lax
from jax.experimental import pallas as pl
from jax.experimental.pallas import tpu as pltpu
from jax.experimental.pallas import tpu_sc as plsc

_NV = 10000
_NCK = 5000
_NE = 160000
_NB = 32
_NT = 5

_NC = 2
_NS = 16
_NW = _NC * _NS

_EPT = _NE // _NW
_CK = 40
_NCH = _EPT // _CK
_CKP = 5120
_CPT = _CKP // _NS
_TBC = 80
_VNCH = _NV // _TBC
_VROUND = 8

_LN2 = 0.6931471805599453
_F32 = jnp.float32
_I32 = jnp.int32

_mesh = plsc.VectorSubcoreMesh(
    core_axis_name="c", subcore_axis_name="s", num_cores=_NC, num_subcores=_NS
)
_params = pltpu.CompilerParams(use_tc_tiling_on_sc=False, needs_layout_passes=False)


_MBITS = 7
_TSHIFT = 23 - _MBITS
_TBASE = 103 << _MBITS
_NSEG = ((131 + 1) << _MBITS) - _TBASE


def _build_phi_table():
    edges_bits = (_np.arange(_NSEG + 1, dtype=_np.int64) + _TBASE) << _TSHIFT
    edges = edges_bits.astype(_np.int32).view(_np.float32).astype(_np.float64)
    y = -_np.log(_np.tanh(edges / 2.0))
    b = (y[1:] - y[:-1]) / (edges[1:] - edges[:-1])
    a = y[:-1] - b * edges[:-1]
    mid = 0.5 * (edges[:-1] + edges[1:])
    a -= ((a + b * mid) - (-_np.log(_np.tanh(mid / 2.0)))) / 2
    return a.astype(_np.float32), b.astype(_np.float32)


_PHI_A, _PHI_B = _build_phi_table()


def _phi(mag, ta, tb):
    bits = lax.bitcast_convert_type(mag, _I32)
    idx = (bits >> _TSHIFT) - _TBASE
    a = plsc.load_gather(ta, [idx])
    b = plsc.load_gather(tb, [idx])
    u = jnp.exp(-mag)
    u2 = u * u
    tail = 2.0 * u * (1.0 + u2 * (1.0 / 3.0 + u2 * 0.2))
    return jnp.where(mag > 3.0, tail, a + b * mag)


def _for(n, body):
    lax.fori_loop(0, n, lambda i, c: (body(i), 0)[1], 0)


def _zero_fill(ref, rows, groups):
    z = jnp.zeros((16,), _F32)

    def row(r):
        for j in range(groups):
            ref[r, pl.ds(16 * j, 16)] = z

    _for(rows, row)


def _make_phase_b(first):
    out_type = [
        jax.ShapeDtypeStruct((_NE, _NB), _F32),
        jax.ShapeDtypeStruct((_NE, _NB), _F32),
        jax.ShapeDtypeStruct((_NC, _CKP, 64), _F32),
    ]
    if not first:
        out_type.append(jax.ShapeDtypeStruct((_NV, _NB), _F32))

    scratch = [
        pltpu.VMEM_SHARED((_CKP, 64), _F32),
        pltpu.VMEM((16,), _F32),
        pltpu.VMEM((_NSEG,), _F32),
        pltpu.VMEM((_NSEG,), _F32),
        pltpu.VMEM((_TBC, 64), _F32),
        pltpu.VMEM((_TBC, _NB), _F32),
        pltpu.VMEM((_TBC, _NB), _F32),
        pltpu.VMEM((_TBC, _NB), _F32),
    ] + 2 * [
        pltpu.VMEM((_CK,), _I32),
        pltpu.VMEM((_CK,), _I32),
        pltpu.VMEM((_CK, _NB), _F32),
        pltpu.VMEM((_CK, _NB), _F32),
        pltpu.VMEM((_CK, _NB), _F32),
        pltpu.VMEM((_CK, _NB), _F32),
        pltpu.VMEM((_CK, _NB), _F32),
        pltpu.VMEM((_CK, 64), _F32),
        pltpu.SemaphoreType.DMA,
        pltpu.SemaphoreType.DMA,
        pltpu.SemaphoreType.DMA,
        pltpu.SemaphoreType.DMA,
    ]

    def body(*refs):
        if first:
            (gsrc, gvec, atab, btab, vidx, cidx,
             v2c_out, sph_out, q_out,
             chk_tab, gv, ta, tb, zb, tb0, tb1, tb2, *dual) = refs
            c2v_in = v2c_in = chn = p_in = None
        else:
            (chn, p_in, gvec, atab, btab, vidx, cidx, c2v_in, v2c_in,
             v2c_out, sph_out, q_out, out_prev,
             chk_tab, gv, ta, tb, zb, tb0, tb1, tb2, *dual) = refs
            gsrc = out_prev
        bufs = (tuple(dual[:12]), tuple(dual[12:]))

        cid = lax.axis_index("c")
        sid = lax.axis_index("s")
        wid = sid * _NC + cid

        pltpu.sync_copy(gvec, gv)
        gamma = gv[...]
        pltpu.sync_copy(atab, ta)
        pltpu.sync_copy(btab, tb)

        if not first:
            def build(i):
                c = sid + i * _NS

                @pl.when(c < _VNCH)
                def _():
                    rows = pl.ds(c * _TBC, _TBC)
                    pltpu.sync_copy(chn.at[rows], tb0)
                    pltpu.sync_copy(p_in.at[0, rows], tb1)
                    pltpu.sync_copy(p_in.at[1, rows], tb2)

                    def addrow(r):
                        for j in range(2):
                            d = pl.ds(16 * j, 16)
                            tb0[r, d] = tb0[r, d] + tb1[r, d] + tb2[r, d]

                    _for(_TBC, addrow)
                    pltpu.sync_copy(tb0, out_prev.at[rows])

            _for(_VROUND, build)

        _zero_fill(zb, _TBC, 4)

        def zchunk(i):
            pltpu.sync_copy(zb, chk_tab.at[pl.ds(sid * _CPT + i * _TBC, _TBC)])

        _for(_CPT // _TBC, zchunk)
        plsc.subcore_barrier()

        def issue_loads(k, B):
            (bvi, bci, bc2v, bv2c, _bg, _bn, _bsp, _bsc, svi, sin, _sg, _sst) = B
            base = wid * _EPT + k * _CK
            pltpu.async_copy(vidx.at[pl.ds(base, _CK)], bvi, svi)
            pltpu.async_copy(cidx.at[pl.ds(base, _CK)], bci, sin)
            if not first:
                pltpu.async_copy(c2v_in.at[pl.ds(base, _CK)], bc2v, sin)
                pltpu.async_copy(v2c_in.at[pl.ds(base, _CK)], bv2c, sin)

        def step(k, P, Q):
            (bvi, bci, bc2v, bv2c, bg, bn, bsp, bsc, svi, sin, sg, sst) = P
            base = wid * _EPT + k * _CK

            @pl.when(k + 1 < _NCH)
            def _():
                issue_loads(k + 1, Q)

            pltpu.make_async_copy(vidx.at[pl.ds(base, _CK)], bvi, svi).wait()
            cpg = pltpu.async_copy(gsrc.at[bvi], bg, sg)
            pltpu.make_async_copy(cidx.at[pl.ds(base, _CK)], bci, sin).wait()
            if not first:
                pltpu.make_async_copy(c2v_in.at[pl.ds(base, _CK)], bc2v, sin).wait()
                pltpu.make_async_copy(v2c_in.at[pl.ds(base, _CK)], bv2c, sin).wait()
            cpg.wait()

            def row(r):
                for j in range(2):
                    d = pl.ds(16 * j, 16)
                    g = bg[r, d]
                    if first:
                        nv = gamma * g
                    else:
                        nv = gamma * (g - bc2v[r, d]) + (1.0 - gamma) * bv2c[r, d]
                    bn[r, d] = nv
                    mag = jnp.clip(jnp.abs(nv), 1e-7, 20.0)
                    ph = _phi(mag, ta, tb)
                    isneg = nv < 0.0
                    sbits = jnp.where(isneg, jnp.int32(-2147483648), jnp.int32(0))
                    bsp[r, d] = lax.bitcast_convert_type(
                        lax.bitcast_convert_type(ph, _I32) | sbits, _F32)
                    bsc[r, d] = ph
                    bsc[r, pl.ds(32 + 16 * j, 16)] = jnp.where(isneg, 1.0, 0.0)

            _for(_CK, row)
            st0 = pltpu.async_copy(bn, v2c_out.at[pl.ds(base, _CK)], sst)
            st1 = pltpu.async_copy(bsp, sph_out.at[pl.ds(base, _CK)], sst)
            pltpu.sync_copy(bsc, chk_tab.at[bci], add=True)
            st0.wait()
            st1.wait()

        issue_loads(0, bufs[0])

        def pair(j):
            step(2 * j, bufs[0], bufs[1])

            @pl.when(2 * j + 1 < _NCH)
            def _():
                step(2 * j + 1, bufs[1], bufs[0])

        _for((_NCH + 1) // 2, pair)
        plsc.subcore_barrier()

        def dchunk(i):
            rows = pl.ds(sid * _CPT + i * _TBC, _TBC)
            pltpu.sync_copy(chk_tab.at[rows], zb)
            pltpu.sync_copy(zb, q_out.at[cid, rows])

        _for(_CPT // _TBC, dchunk)

    return pl.kernel(body, out_type=out_type, mesh=_mesh, scratch_types=scratch,
                     compiler_params=_params, name="bp_phase_b0" if first else "bp_phase_b")


def _make_phase_c(first):
    out_type = [
        jax.ShapeDtypeStruct((_NE, _NB), _F32),
        jax.ShapeDtypeStruct((_NC, _NV, _NB), _F32),
        jax.ShapeDtypeStruct((_CKP, 64), _F32),
    ]
    scratch = [
        pltpu.VMEM_SHARED((_NV, _NB), _F32),
        pltpu.VMEM((16,), _F32),
        pltpu.VMEM((_NSEG,), _F32),
        pltpu.VMEM((_NSEG,), _F32),
        pltpu.VMEM((_TBC, 64), _F32),
        pltpu.VMEM((_TBC, 64), _F32),
        pltpu.VMEM((_TBC, _NB), _F32),
    ] + 2 * [
        pltpu.VMEM((_CK,), _I32),
        pltpu.VMEM((_CK,), _I32),
        pltpu.VMEM((_CK, _NB), _F32),
        pltpu.VMEM((_CK, _NB), _F32),
        pltpu.VMEM((_CK, _NB), _F32),
        pltpu.VMEM((_CK, 64), _F32),
        pltpu.SemaphoreType.DMA,
        pltpu.SemaphoreType.DMA,
        pltpu.SemaphoreType.DMA,
        pltpu.SemaphoreType.DMA,
    ]

    def body(*refs):
        if first:
            (gvec, atab, btab, vidx, cidx, sph_in, q_in,
             c2v_out, p_out, chkc,
             p_tab, gv, ta, tb, tb0, tb1, zb, *dual) = refs
            c2v_in = None
        else:
            (gvec, atab, btab, vidx, cidx, sph_in, c2v_in, q_in,
             c2v_out, p_out, chkc,
             p_tab, gv, ta, tb, tb0, tb1, zb, *dual) = refs
        bufs = (tuple(dual[:10]), tuple(dual[10:]))

        cid = lax.axis_index("c")
        sid = lax.axis_index("s")
        wid = sid * _NC + cid

        pltpu.sync_copy(gvec, gv)
        gamma = gv[...]
        pltpu.sync_copy(atab, ta)
        pltpu.sync_copy(btab, tb)

        def cchunk(i):
            crows = pl.ds(sid * _CPT + i * _TBC, _TBC)
            pltpu.sync_copy(q_in.at[0, crows], tb0)
            pltpu.sync_copy(q_in.at[1, crows], tb1)

            def addrow(r):
                for j in range(4):
                    d = pl.ds(16 * j, 16)
                    tb0[r, d] = tb0[r, d] + tb1[r, d]

            _for(_TBC, addrow)
            pltpu.sync_copy(tb0, chkc.at[crows])

        _for(_CPT // _TBC, cchunk)

        _zero_fill(zb, _TBC, 2)

        def zchunk(i):
            c = sid + i * _NS

            @pl.when(c < _VNCH)
            def _():
                pltpu.sync_copy(zb, p_tab.at[pl.ds(c * _TBC, _TBC)])

        _for(_VROUND, zchunk)
        plsc.subcore_barrier()

        def issue_loads(k, B):
            (bci, bvi, bsp, bc2v, _bn, _bg, sci, sin, _sg, _sst) = B
            base = wid * _EPT + k * _CK
            pltpu.async_copy(cidx.at[pl.ds(base, _CK)], bci, sci)
            pltpu.async_copy(vidx.at[pl.ds(base, _CK)], bvi, sin)
            pltpu.async_copy(sph_in.at[pl.ds(base, _CK)], bsp, sin)
            if not first:
                pltpu.async_copy(c2v_in.at[pl.ds(base, _CK)], bc2v, sin)

        def step(k, P, Q):
            (bci, bvi, bsp, bc2v, bn, bg, sci, sin, sg, sst) = P
            base = wid * _EPT + k * _CK

            @pl.when(k + 1 < _NCH)
            def _():
                issue_loads(k + 1, Q)

            pltpu.make_async_copy(cidx.at[pl.ds(base, _CK)], bci, sci).wait()
            cpg = pltpu.async_copy(chkc.at[bci], bg, sg)
            pltpu.make_async_copy(vidx.at[pl.ds(base, _CK)], bvi, sin).wait()
            pltpu.make_async_copy(sph_in.at[pl.ds(base, _CK)], bsp, sin).wait()
            if not first:
                pltpu.make_async_copy(c2v_in.at[pl.ds(base, _CK)], bc2v, sin).wait()
            cpg.wait()

            def row(r):
                for j in range(2):
                    d = pl.ds(16 * j, 16)
                    sph = bsp[r, d]
                    bits = lax.bitcast_convert_type(sph, _I32)
                    ph = jnp.abs(sph)
                    negf = jnp.where(bits < 0, 1.0, 0.0)
                    phs = bg[r, d]
                    ns = bg[r, pl.ds(32 + 16 * j, 16)]
                    excl = jnp.clip(phs - ph, 1e-7, 30.0)
                    nm = _phi(excl, ta, tb)
                    par = ((ns - negf).astype(_I32) & 1).astype(_F32)
                    sgn = 1.0 - 2.0 * par
                    if first:
                        nc = gamma * (sgn * nm)
                    else:
                        nc = gamma * (sgn * nm) + (1.0 - gamma) * bc2v[r, d]
                    bn[r, d] = nc

            _for(_CK, row)
            st0 = pltpu.async_copy(bn, c2v_out.at[pl.ds(base, _CK)], sst)
            pltpu.sync_copy(bn, p_tab.at[bvi], add=True)
            st0.wait()

        issue_loads(0, bufs[0])

        def pair(j):
            step(2 * j, bufs[0], bufs[1])

            @pl.when(2 * j + 1 < _NCH)
            def _():
                step(2 * j + 1, bufs[1], bufs[0])

        _for((_NCH + 1) // 2, pair)
        plsc.subcore_barrier()

        def dchunk(i):
            c = sid + i * _NS

            @pl.when(c < _VNCH)
            def _():
                vrows = pl.ds(c * _TBC, _TBC)
                pltpu.sync_copy(p_tab.at[vrows], zb)
                pltpu.sync_copy(zb, p_out.at[cid, vrows])

        _for(_VROUND, dchunk)

    return pl.kernel(body, out_type=out_type, mesh=_mesh, scratch_types=scratch,
                     compiler_params=_params, name="bp_phase_c0" if first else "bp_phase_c")


def _make_combine():
    out_type = jax.ShapeDtypeStruct((_NV, _NB), _F32)
    scratch = [
        pltpu.VMEM((_TBC, _NB), _F32),
        pltpu.VMEM((_TBC, _NB), _F32),
        pltpu.VMEM((_TBC, _NB), _F32),
    ]

    def body(chn, p_in, out, tb0, tb1, tb2):
        cid = lax.axis_index("c")
        sid = lax.axis_index("s")
        wid = sid * _NC + cid

        def build(i):
            c = wid + i * _NW

            @pl.when(c < _VNCH)
            def _():
                rows = pl.ds(c * _TBC, _TBC)
                pltpu.sync_copy(chn.at[rows], tb0)
                pltpu.sync_copy(p_in.at[0, rows], tb1)
                pltpu.sync_copy(p_in.at[1, rows], tb2)

                def addrow(r):
                    for j in range(2):
                        d = pl.ds(16 * j, 16)
                        tb0[r, d] = tb0[r, d] + tb1[r, d] + tb2[r, d]

                _for(_TBC, addrow)
                pltpu.sync_copy(tb0, out.at[rows])

        _for(4, build)

    return pl.kernel(body, out_type=out_type, mesh=_mesh, scratch_types=scratch,
                     compiler_params=_params, name="bp_combine")


_phase_b_first = _make_phase_b(True)
_phase_b_rest = _make_phase_b(False)
_phase_c_first = _make_phase_c(True)
_phase_c_rest = _make_phase_c(False)
_combine = _make_combine()


def kernel(chn_llr, gamma_logit, var_idx, chk_idx):
    gvec = jnp.full((16,), jax.nn.sigmoid(gamma_logit[0]), dtype=_F32)
    ta = jnp.asarray(_PHI_A)
    tb = jnp.asarray(_PHI_B)

    v2c, sph, q = _phase_b_first(chn_llr, gvec, ta, tb, var_idx, chk_idx)
    c2v, p, _unused = _phase_c_first(gvec, ta, tb, var_idx, chk_idx, sph, q)

    outs = []
    for _ in range(_NT - 1):
        v2c, sph, q, out_prev = _phase_b_rest(
            chn_llr, p, gvec, ta, tb, var_idx, chk_idx, c2v, v2c)
        outs.append(out_prev)
        c2v, p, _unused = _phase_c_rest(gvec, ta, tb, var_idx, chk_idx, sph, c2v, q)

    outs.append(_combine(chn_llr, p))
    return tuple(outs)


# speedup vs baseline: 1.0115x; 1.0115x over previous
import jax
import jax.numpy as jnp
import numpy as _np
from jax import lax
from jax.experimental import pallas as pl
from jax.experimental.pallas import tpu as pltpu
from jax.experimental.pallas import tpu_sc as plsc

_NV = 10000
_NCK = 5000
_NE = 160000
_NB = 32
_NT = 5

_NC = 2
_NS = 16
_NW = _NC * _NS

_EPT = _NE // _NW
_CK = 40
_NCH = _EPT // _CK
_CKP = 5120
_CPT = _CKP // _NS
_TBC = 80
_VNCH = _NV // _TBC
_VROUND = 8

_LN2 = 0.6931471805599453
_F32 = jnp.float32
_I32 = jnp.int32

_mesh = plsc.VectorSubcoreMesh(
    core_axis_name="c", subcore_axis_name="s", num_cores=_NC, num_subcores=_NS
)
_params = pltpu.CompilerParams(use_tc_tiling_on_sc=False, needs_layout_passes=False)


_MBITS = 7
_TSHIFT = 23 - _MBITS
_TBASE = 103 << _MBITS
_NSEG = ((131 + 1) << _MBITS) - _TBASE


def _build_phi_table():
    edges_bits = (_np.arange(_NSEG + 1, dtype=_np.int64) + _TBASE) << _TSHIFT
    edges = edges_bits.astype(_np.int32).view(_np.float32).astype(_np.float64)
    y = -_np.log(_np.tanh(edges / 2.0))
    b = (y[1:] - y[:-1]) / (edges[1:] - edges[:-1])
    a = y[:-1] - b * edges[:-1]
    mid = 0.5 * (edges[:-1] + edges[1:])
    a -= ((a + b * mid) - (-_np.log(_np.tanh(mid / 2.0)))) / 2
    return a.astype(_np.float32), b.astype(_np.float32)


_PHI_A, _PHI_B = _build_phi_table()


def _phi(mag, ta, tb):
    bits = lax.bitcast_convert_type(mag, _I32)
    idx = (bits >> _TSHIFT) - _TBASE
    a = plsc.load_gather(ta, [idx])
    b = plsc.load_gather(tb, [idx])
    u = jnp.exp(-mag)
    u2 = u * u
    tail = 2.0 * u * (1.0 + u2 * (1.0 / 3.0 + u2 * 0.2))
    return jnp.where(mag > 3.0, tail, a + b * mag)


def _for(n, body):
    lax.fori_loop(0, n, lambda i, c: (body(i), 0)[1], 0)


def _zero_fill(ref, rows, groups):
    z = jnp.zeros((16,), _F32)

    def row(r):
        for j in range(groups):
            ref[r, pl.ds(16 * j, 16)] = z

    _for(rows, row)


def _make_phase_b(first):
    out_type = [
        jax.ShapeDtypeStruct((_NE, _NB), _F32),
        jax.ShapeDtypeStruct((_NE, _NB), _F32),
        jax.ShapeDtypeStruct((_NC, _CKP, 64), _F32),
    ]

    scratch = [
        pltpu.VMEM_SHARED((_CKP, 64), _F32),
        pltpu.VMEM((16,), _F32),
        pltpu.VMEM((_NSEG,), _F32),
        pltpu.VMEM((_NSEG,), _F32),
        pltpu.VMEM((_TBC, 64), _F32),
    ] + 2 * [
        pltpu.VMEM((_CK,), _I32),
        pltpu.VMEM((_CK,), _I32),
        pltpu.VMEM((_CK, _NB), _F32),
        pltpu.VMEM((_CK, _NB), _F32),
        pltpu.VMEM((_CK, _NB), _F32),
        pltpu.VMEM((_CK, _NB), _F32),
        pltpu.VMEM((_CK, _NB), _F32),
        pltpu.VMEM((_CK, 64), _F32),
        pltpu.SemaphoreType.DMA,
        pltpu.SemaphoreType.DMA,
        pltpu.SemaphoreType.DMA,
        pltpu.SemaphoreType.DMA,
    ]

    def body(*refs):
        if first:
            (gsrc, gvec, atab, btab, vidx, cidx,
             v2c_out, sph_out, q_out,
             chk_tab, gv, ta, tb, zb, *dual) = refs
            c2v_in = v2c_in = None
        else:
            (gsrc, gvec, atab, btab, vidx, cidx, c2v_in, v2c_in,
             v2c_out, sph_out, q_out,
             chk_tab, gv, ta, tb, zb, *dual) = refs
        bufs = (tuple(dual[:12]), tuple(dual[12:]))

        cid = lax.axis_index("c")
        sid = lax.axis_index("s")
        wid = sid * _NC + cid

        pltpu.sync_copy(gvec, gv)
        gamma = gv[...]
        pltpu.sync_copy(atab, ta)
        pltpu.sync_copy(btab, tb)

        _zero_fill(zb, _TBC, 4)

        def zchunk(i):
            pltpu.sync_copy(zb, chk_tab.at[pl.ds(sid * _CPT + i * _TBC, _TBC)])

        _for(_CPT // _TBC, zchunk)
        plsc.subcore_barrier()

        def issue_loads(k, B):
            (bvi, bci, bc2v, bv2c, _bg, _bn, _bsp, _bsc, svi, sin, _sg, _sst) = B
            base = wid * _EPT + k * _CK
            pltpu.async_copy(vidx.at[pl.ds(base, _CK)], bvi, svi)
            pltpu.async_copy(cidx.at[pl.ds(base, _CK)], bci, sin)
            if not first:
                pltpu.async_copy(c2v_in.at[pl.ds(base, _CK)], bc2v, sin)
                pltpu.async_copy(v2c_in.at[pl.ds(base, _CK)], bv2c, sin)

        def step(k, P, Q):
            (bvi, bci, bc2v, bv2c, bg, bn, bsp, bsc, svi, sin, sg, sst) = P
            base = wid * _EPT + k * _CK

            @pl.when(k + 1 < _NCH)
            def _():
                issue_loads(k + 1, Q)

            pltpu.make_async_copy(vidx.at[pl.ds(base, _CK)], bvi, svi).wait()
            cpg = pltpu.async_copy(gsrc.at[bvi], bg, sg)
            pltpu.make_async_copy(cidx.at[pl.ds(base, _CK)], bci, sin).wait()
            if not first:
                pltpu.make_async_copy(c2v_in.at[pl.ds(base, _CK)], bc2v, sin).wait()
                pltpu.make_async_copy(v2c_in.at[pl.ds(base, _CK)], bv2c, sin).wait()
            cpg.wait()

            def row(r):
                for j in range(2):
                    d = pl.ds(16 * j, 16)
                    g = bg[r, d]
                    if first:
                        nv = gamma * g
                    else:
                        nv = gamma * (g - bc2v[r, d]) + (1.0 - gamma) * bv2c[r, d]
                    bn[r, d] = nv
                    mag = jnp.clip(jnp.abs(nv), 1e-7, 20.0)
                    ph = _phi(mag, ta, tb)
                    isneg = nv < 0.0
                    sbits = jnp.where(isneg, jnp.int32(-2147483648), jnp.int32(0))
                    bsp[r, d] = lax.bitcast_convert_type(
                        lax.bitcast_convert_type(ph, _I32) | sbits, _F32)
                    bsc[r, d] = ph
                    bsc[r, pl.ds(32 + 16 * j, 16)] = jnp.where(isneg, 1.0, 0.0)

            def row2(r2):
                row(2 * r2)
                row(2 * r2 + 1)

            _for(_CK // 2, row2)
            st0 = pltpu.async_copy(bn, v2c_out.at[pl.ds(base, _CK)], sst)
            st1 = pltpu.async_copy(bsp, sph_out.at[pl.ds(base, _CK)], sst)
            pltpu.sync_copy(bsc, chk_tab.at[bci], add=True)
            st0.wait()
            st1.wait()

        issue_loads(0, bufs[0])

        def pair(j):
            step(2 * j, bufs[0], bufs[1])

            @pl.when(2 * j + 1 < _NCH)
            def _():
                step(2 * j + 1, bufs[1], bufs[0])

        _for((_NCH + 1) // 2, pair)
        plsc.subcore_barrier()

        def dchunk(i):
            rows = pl.ds(sid * _CPT + i * _TBC, _TBC)
            pltpu.sync_copy(chk_tab.at[rows], zb)
            pltpu.sync_copy(zb, q_out.at[cid, rows])

        _for(_CPT // _TBC, dchunk)

    return pl.kernel(body, out_type=out_type, mesh=_mesh, scratch_types=scratch,
                     compiler_params=_params, name="bp_phase_b0" if first else "bp_phase_b")


def _make_phase_c(first):
    out_type = [
        jax.ShapeDtypeStruct((_NE, _NB), _F32),
        jax.ShapeDtypeStruct((_NC, _NV, _NB), _F32),
        jax.ShapeDtypeStruct((_CKP, 64), _F32),
    ]
    scratch = [
        pltpu.VMEM_SHARED((_NV, _NB), _F32),
        pltpu.VMEM((16,), _F32),
        pltpu.VMEM((_NSEG,), _F32),
        pltpu.VMEM((_NSEG,), _F32),
        pltpu.VMEM((_TBC, 64), _F32),
        pltpu.VMEM((_TBC, 64), _F32),
        pltpu.VMEM((_TBC, _NB), _F32),
    ] + 2 * [
        pltpu.VMEM((_CK,), _I32),
        pltpu.VMEM((_CK,), _I32),
        pltpu.VMEM((_CK, _NB), _F32),
        pltpu.VMEM((_CK, _NB), _F32),
        pltpu.VMEM((_CK, _NB), _F32),
        pltpu.VMEM((_CK, 64), _F32),
        pltpu.SemaphoreType.DMA,
        pltpu.SemaphoreType.DMA,
        pltpu.SemaphoreType.DMA,
        pltpu.SemaphoreType.DMA,
    ]

    def body(*refs):
        if first:
            (gvec, atab, btab, vidx, cidx, sph_in, q_in,
             c2v_out, p_out, chkc,
             p_tab, gv, ta, tb, tb0, tb1, zb, *dual) = refs
            c2v_in = None
        else:
            (gvec, atab, btab, vidx, cidx, sph_in, c2v_in, q_in,
             c2v_out, p_out, chkc,
             p_tab, gv, ta, tb, tb0, tb1, zb, *dual) = refs
        bufs = (tuple(dual[:10]), tuple(dual[10:]))

        cid = lax.axis_index("c")
        sid = lax.axis_index("s")
        wid = sid * _NC + cid

        pltpu.sync_copy(gvec, gv)
        gamma = gv[...]
        pltpu.sync_copy(atab, ta)
        pltpu.sync_copy(btab, tb)

        def cchunk(i):
            crows = pl.ds(sid * _CPT + i * _TBC, _TBC)
            pltpu.sync_copy(q_in.at[0, crows], tb0)
            pltpu.sync_copy(q_in.at[1, crows], tb1)

            def addrow(r):
                for j in range(4):
                    d = pl.ds(16 * j, 16)
                    tb0[r, d] = tb0[r, d] + tb1[r, d]

            _for(_TBC, addrow)
            pltpu.sync_copy(tb0, chkc.at[crows])

        _for(_CPT // _TBC, cchunk)

        _zero_fill(zb, _TBC, 2)

        def zchunk(i):
            c = sid + i * _NS

            @pl.when(c < _VNCH)
            def _():
                pltpu.sync_copy(zb, p_tab.at[pl.ds(c * _TBC, _TBC)])

        _for(_VROUND, zchunk)
        plsc.subcore_barrier()

        def issue_loads(k, B):
            (bci, bvi, bsp, bc2v, _bn, _bg, sci, sin, _sg, _sst) = B
            base = wid * _EPT + k * _CK
            pltpu.async_copy(cidx.at[pl.ds(base, _CK)], bci, sci)
            pltpu.async_copy(vidx.at[pl.ds(base, _CK)], bvi, sin)
            pltpu.async_copy(sph_in.at[pl.ds(base, _CK)], bsp, sin)
            if not first:
                pltpu.async_copy(c2v_in.at[pl.ds(base, _CK)], bc2v, sin)

        def step(k, P, Q):
            (bci, bvi, bsp, bc2v, bn, bg, sci, sin, sg, sst) = P
            base = wid * _EPT + k * _CK

            @pl.when(k + 1 < _NCH)
            def _():
                issue_loads(k + 1, Q)

            pltpu.make_async_copy(cidx.at[pl.ds(base, _CK)], bci, sci).wait()
            cpg = pltpu.async_copy(chkc.at[bci], bg, sg)
            pltpu.make_async_copy(vidx.at[pl.ds(base, _CK)], bvi, sin).wait()
            pltpu.make_async_copy(sph_in.at[pl.ds(base, _CK)], bsp, sin).wait()
            if not first:
                pltpu.make_async_copy(c2v_in.at[pl.ds(base, _CK)], bc2v, sin).wait()
            cpg.wait()

            def row(r):
                for j in range(2):
                    d = pl.ds(16 * j, 16)
                    sph = bsp[r, d]
                    bits = lax.bitcast_convert_type(sph, _I32)
                    ph = jnp.abs(sph)
                    negf = jnp.where(bits < 0, 1.0, 0.0)
                    phs = bg[r, d]
                    ns = bg[r, pl.ds(32 + 16 * j, 16)]
                    excl = jnp.clip(phs - ph, 1e-7, 30.0)
                    nm = _phi(excl, ta, tb)
                    par = ((ns - negf).astype(_I32) & 1).astype(_F32)
                    sgn = 1.0 - 2.0 * par
                    if first:
                        nc = gamma * (sgn * nm)
                    else:
                        nc = gamma * (sgn * nm) + (1.0 - gamma) * bc2v[r, d]
                    bn[r, d] = nc

            def row2(r2):
                row(2 * r2)
                row(2 * r2 + 1)

            _for(_CK // 2, row2)
            st0 = pltpu.async_copy(bn, c2v_out.at[pl.ds(base, _CK)], sst)
            pltpu.sync_copy(bn, p_tab.at[bvi], add=True)
            st0.wait()

        issue_loads(0, bufs[0])

        def pair(j):
            step(2 * j, bufs[0], bufs[1])

            @pl.when(2 * j + 1 < _NCH)
            def _():
                step(2 * j + 1, bufs[1], bufs[0])

        _for((_NCH + 1) // 2, pair)
        plsc.subcore_barrier()

        def dchunk(i):
            c = sid + i * _NS

            @pl.when(c < _VNCH)
            def _():
                vrows = pl.ds(c * _TBC, _TBC)
                pltpu.sync_copy(p_tab.at[vrows], zb)
                pltpu.sync_copy(zb, p_out.at[cid, vrows])

        _for(_VROUND, dchunk)

    return pl.kernel(body, out_type=out_type, mesh=_mesh, scratch_types=scratch,
                     compiler_params=_params, name="bp_phase_c0" if first else "bp_phase_c")


def _make_combine():
    out_type = jax.ShapeDtypeStruct((_NV, _NB), _F32)
    scratch = [
        pltpu.VMEM((_TBC, _NB), _F32),
        pltpu.VMEM((_TBC, _NB), _F32),
        pltpu.VMEM((_TBC, _NB), _F32),
    ]

    def body(chn, p_in, out, tb0, tb1, tb2):
        cid = lax.axis_index("c")
        sid = lax.axis_index("s")
        wid = sid * _NC + cid

        def build(i):
            c = wid + i * _NW

            @pl.when(c < _VNCH)
            def _():
                rows = pl.ds(c * _TBC, _TBC)
                pltpu.sync_copy(chn.at[rows], tb0)
                pltpu.sync_copy(p_in.at[0, rows], tb1)
                pltpu.sync_copy(p_in.at[1, rows], tb2)

                def addrow(r):
                    for j in range(2):
                        d = pl.ds(16 * j, 16)
                        tb0[r, d] = tb0[r, d] + tb1[r, d] + tb2[r, d]

                _for(_TBC, addrow)
                pltpu.sync_copy(tb0, out.at[rows])

        _for(4, build)

    return pl.kernel(body, out_type=out_type, mesh=_mesh, scratch_types=scratch,
                     compiler_params=_params, name="bp_combine")


_phase_b_first = _make_phase_b(True)
_phase_b_rest = _make_phase_b(False)
_phase_c_first = _make_phase_c(True)
_phase_c_rest = _make_phase_c(False)
_combine = _make_combine()


def kernel(chn_llr, gamma_logit, var_idx, chk_idx):
    gvec = jnp.full((16,), jax.nn.sigmoid(gamma_logit[0]), dtype=_F32)
    ta = jnp.asarray(_PHI_A)
    tb = jnp.asarray(_PHI_B)

    v2c, sph, q = _phase_b_first(chn_llr, gvec, ta, tb, var_idx, chk_idx)
    c2v, p, _unused = _phase_c_first(gvec, ta, tb, var_idx, chk_idx, sph, q)

    outs = []
    for _ in range(_NT - 1):
        g = _combine(chn_llr, p)
        outs.append(g)
        v2c, sph, q = _phase_b_rest(g, gvec, ta, tb, var_idx, chk_idx, c2v, v2c)
        c2v, p, _unused = _phase_c_rest(gvec, ta, tb, var_idx, chk_idx, sph, c2v, q)

    outs.append(_combine(chn_llr, p))
    return tuple(outs)
